# Initial kernel scaffold; baseline (speedup 1.0000x reference)
import functools

import jax
import jax.numpy as jnp
from jax.experimental import pallas as pl
from jax.experimental.pallas import tpu as pltpu


def _mm_body(x_ref, w_ref, b_ref, o_ref, *, act):
    acc = jnp.dot(x_ref[...], w_ref[...], preferred_element_type=jnp.float32)
    acc = acc + b_ref[...]
    if act == "relu":
        acc = jnp.maximum(acc, 0.0)
    o_ref[...] = acc


def _mm(x, w, b, act="none", block_m=1024):
    m, k = x.shape
    n = w.shape[1]
    bm = min(block_m, m)
    grid = (pl.cdiv(m, bm),)
    return pl.pallas_call(
        functools.partial(_mm_body, act=act),
        grid=grid,
        in_specs=[
            pl.BlockSpec((bm, k), lambda i: (i, 0)),
            pl.BlockSpec((k, n), lambda i: (0, 0)),
            pl.BlockSpec((1, n), lambda i: (0, 0)),
        ],
        out_specs=pl.BlockSpec((bm, n), lambda i: (i, 0)),
        out_shape=jax.ShapeDtypeStruct((m, n), jnp.float32),
    )(x, w, b.reshape(1, n))


def _gcn_agg(xn, ei, dinv):
    src, dst = ei[0], ei[1]
    s = jnp.zeros_like(xn).at[dst].add(xn[src])
    return (s + xn) * dinv[:, None]


def _branch(x, ei, batch, nB, W1, b1, W2, b2, W3, b3, Wg1, bg1, Wg2, bg2):
    n = x.shape[0]
    deg = jnp.ones((n,), jnp.float32).at[ei[1]].add(1.0)
    dinv = jax.lax.rsqrt(deg)
    h = jnp.maximum(_mm(_gcn_agg(x * dinv[:, None], ei, dinv), W1, b1), 0.0)
    h = jnp.maximum(_mm(_gcn_agg(h * dinv[:, None], ei, dinv), W2, b2), 0.0)
    h = jnp.maximum(_mm(_gcn_agg(h * dinv[:, None], ei, dinv), W3, b3), 0.0)
    g = jnp.zeros((nB, h.shape[1]), h.dtype).at[batch].max(h)
    g = _mm(g, Wg1, bg1, act="relu")
    g = _mm(g, Wg2, bg2)
    return g


def kernel(x1, edge_index1, batch1, cell, x2, edge_index2, batch2, W1, b1, W2, b2, W3, b3, Wg1, bg1, Wg2, bg2, Wr1, br1, Wr2, br2, Wr3, br3, Wf1, bf1, Wf2, bf2, Wf3, bf3, Wo, bo):
    nB = cell.shape[0]
    g1 = _branch(x1, edge_index1, batch1, nB, W1, b1, W2, b2, W3, b3, Wg1, bg1, Wg2, bg2)
    g2 = _branch(x2, edge_index2, batch2, nB, W1, b1, W2, b2, W3, b3, Wg1, bg1, Wg2, bg2)
    cv = cell / jnp.maximum(jnp.linalg.norm(cell, axis=1, keepdims=True), 1e-12)
    cv = _mm(cv, Wr1, br1, act="relu")
    cv = _mm(cv, Wr2, br2, act="relu")
    cv = _mm(cv, Wr3, br3, act="relu")
    xc = jnp.concatenate([g1, g2, cv], axis=1)
    xc = _mm(xc, Wf1, bf1, act="relu")
    xc = _mm(xc, Wf2, bf2, act="relu")
    xc = _mm(xc, Wf3, bf3, act="relu")
    return _mm(xc, Wo, bo)


# speedup vs baseline: 3.6988x; 3.6988x over previous
import functools

import jax
import jax.numpy as jnp
from jax.experimental import pallas as pl
from jax.experimental.pallas import tpu as pltpu


def _mm_body(x_ref, w_ref, b_ref, o_ref, *, act):
    acc = jnp.dot(x_ref[...], w_ref[...], preferred_element_type=jnp.float32,
                  precision=jax.lax.Precision.HIGHEST)
    acc = acc + b_ref[...]
    if act == "relu":
        acc = jnp.maximum(acc, 0.0)
    o_ref[...] = acc


def _mm(x, w, b, act="none", block_m=1024):
    m, k = x.shape
    n = w.shape[1]
    bm = min(block_m, m)
    grid = (pl.cdiv(m, bm),)
    return pl.pallas_call(
        functools.partial(_mm_body, act=act),
        grid=grid,
        in_specs=[
            pl.BlockSpec((bm, k), lambda i: (i, 0)),
            pl.BlockSpec((k, n), lambda i: (0, 0)),
            pl.BlockSpec((1, n), lambda i: (0, 0)),
        ],
        out_specs=pl.BlockSpec((bm, n), lambda i: (i, 0)),
        out_shape=jax.ShapeDtypeStruct((m, n), jnp.float32),
    )(x, w, b.reshape(1, n))


def _gcn_agg(xn, ei, dinv):
    src, dst = ei[0], ei[1]
    s = jnp.zeros_like(xn).at[dst].add(xn[src])
    return (s + xn) * dinv[:, None]


def _branch(x, ei, batch, nB, W1, b1, W2, b2, W3, b3, Wg1, bg1, Wg2, bg2):
    n = x.shape[0]
    deg = jnp.ones((n,), jnp.float32).at[ei[1]].add(1.0)
    dinv = jax.lax.rsqrt(deg)
    h = jnp.maximum(_mm(_gcn_agg(x * dinv[:, None], ei, dinv), W1, b1), 0.0)
    h = jnp.maximum(_mm(_gcn_agg(h * dinv[:, None], ei, dinv), W2, b2), 0.0)
    h = jnp.maximum(_mm(_gcn_agg(h * dinv[:, None], ei, dinv), W3, b3), 0.0)
    g = jnp.zeros((nB, h.shape[1]), h.dtype).at[batch].max(h)
    g = _mm(g, Wg1, bg1, act="relu")
    g = _mm(g, Wg2, bg2)
    return g


def kernel(x1, edge_index1, batch1, cell, x2, edge_index2, batch2, W1, b1, W2, b2, W3, b3, Wg1, bg1, Wg2, bg2, Wr1, br1, Wr2, br2, Wr3, br3, Wf1, bf1, Wf2, bf2, Wf3, bf3, Wo, bo):
    nB = cell.shape[0]
    g1 = _branch(x1, edge_index1, batch1, nB, W1, b1, W2, b2, W3, b3, Wg1, bg1, Wg2, bg2)
    g2 = _branch(x2, edge_index2, batch2, nB, W1, b1, W2, b2, W3, b3, Wg1, bg1, Wg2, bg2)
    cv = cell / jnp.maximum(jnp.linalg.norm(cell, axis=1, keepdims=True), 1e-12)
    cv = _mm(cv, Wr1, br1, act="relu")
    cv = _mm(cv, Wr2, br2, act="relu")
    cv = _mm(cv, Wr3, br3, act="relu")
    xc = jnp.concatenate([g1, g2, cv], axis=1)
    xc = _mm(xc, Wf1, bf1, act="relu")
    xc = _mm(xc, Wf2, bf2, act="relu")
    xc = _mm(xc, Wf3, bf3, act="relu")
    return _mm(xc, Wo, bo)


# speedup vs baseline: 8.5085x; 2.3003x over previous
import functools

import jax
import jax.numpy as jnp
from jax import lax
from jax.experimental import pallas as pl
from jax.experimental.pallas import tpu as pltpu
from jax.experimental.pallas import tpu_sc as plsc

_NC = 2
_NS = 16
_FC = 32
_G = 8


def _pad_rows(n):
    return pl.cdiv(n, 1024) * 1024


def _nchunks(f):
    return pl.cdiv(f, _FC)


def _sc_degrees(dst2d_1, dst2d_2, ones_col, zeros_col, np_rows):
    er = dst2d_1.shape[0]
    rpt = er // _NS
    ngroups = rpt // _G
    slab = np_rows // _NS
    mesh = plsc.VectorSubcoreMesh(core_axis_name="c", subcore_axis_name="s")

    @functools.partial(
        pl.kernel, mesh=mesh,
        compiler_params=pltpu.CompilerParams(use_tc_tiling_on_sc=False),
        out_type=[jax.ShapeDtypeStruct((np_rows, 1), jnp.float32)] * 2,
        scratch_types=[
            pltpu.VMEM((_G, 128), jnp.int32),
            pltpu.VMEM((128, 1), jnp.float32),
            pltpu.VMEM_SHARED((np_rows, 1), jnp.float32),
        ],
    )
    def k(d1_ref, d2_ref, ones_ref, zeros_ref, o1_ref, o2_ref, idx_v, ones_v, acc):
        cid = lax.axis_index("c")
        sid = lax.axis_index("s")
        pltpu.sync_copy(ones_ref, ones_v)

        def run(dref, oref):
            pltpu.sync_copy(zeros_ref, acc.at[pl.ds(sid * slab, slab)])
            plsc.subcore_barrier()

            def body(g, _):
                base = sid * rpt + g * _G
                pltpu.sync_copy(dref.at[pl.ds(base, _G)], idx_v)
                for j in range(_G):
                    pltpu.sync_copy(ones_v, acc.at[idx_v.at[j]], add=True)
                return 0

            lax.fori_loop(0, ngroups, body, 0)
            plsc.subcore_barrier()
            pltpu.sync_copy(acc.at[pl.ds(sid * slab, slab)],
                            oref.at[pl.ds(sid * slab, slab)])

        @pl.when(cid == 0)
        def _():
            run(d1_ref, o1_ref)

        @pl.when(cid == 1)
        def _():
            run(d2_ref, o2_ref)

    return k(dst2d_1, dst2d_2, ones_col, zeros_col)


def _sc_agg(jobs, zeros_slab, np_rows):
    njobs = len(jobs)
    er = jobs[0][1].shape[0]
    rpt = er // _NS
    ngroups = rpt // _G
    slab = np_rows // _NS
    mesh = plsc.VectorSubcoreMesh(core_axis_name="c", subcore_axis_name="s")

    @functools.partial(
        pl.kernel, mesh=mesh,
        compiler_params=pltpu.CompilerParams(use_tc_tiling_on_sc=False),
        out_type=[jax.ShapeDtypeStruct((np_rows, _FC), jnp.float32)] * njobs,
        scratch_types=[
            pltpu.VMEM((_G, 128), jnp.int32),
            pltpu.VMEM((_G, 128), jnp.int32),
            pltpu.VMEM((2, 128, _FC), jnp.float32),
            pltpu.VMEM_SHARED((np_rows, _FC), jnp.float32),
            pltpu.SemaphoreType.DMA,
        ],
    )
    def k(*refs):
        in_refs = refs[:3 * njobs]
        zeros_ref = refs[3 * njobs]
        out_refs = refs[3 * njobs + 1:3 * njobs + 1 + njobs]
        src_v, dst_v, rows_v, acc, gsem = refs[3 * njobs + 1 + njobs:]
        cid = lax.axis_index("c")
        sid = lax.axis_index("s")

        def sweep(xn_ref, src_ref, dst_ref, o_ref):
            pltpu.sync_copy(zeros_ref, acc.at[pl.ds(sid * slab, slab)])
            plsc.subcore_barrier()

            def body(g, _):
                base = sid * rpt + g * _G
                pltpu.sync_copy(src_ref.at[pl.ds(base, _G)], src_v)
                pltpu.sync_copy(dst_ref.at[pl.ds(base, _G)], dst_v)
                for j in range(_G):
                    buf = rows_v.at[j % 2]
                    pltpu.async_copy(xn_ref.at[src_v.at[j]], buf, gsem).wait()
                    pltpu.sync_copy(buf, acc.at[dst_v.at[j]], add=True)
                return 0

            lax.fori_loop(0, ngroups, body, 0)
            plsc.subcore_barrier()
            pltpu.sync_copy(acc.at[pl.ds(sid * slab, slab)],
                            o_ref.at[pl.ds(sid * slab, slab)])

        for i in range(njobs):
            @pl.when(cid == (i % _NC))
            def _(i=i):
                sweep(in_refs[3 * i], in_refs[3 * i + 1], in_refs[3 * i + 2],
                      out_refs[i])

    flat = [r for job in jobs for r in job]
    return k(*flat, zeros_slab)


def _xn0_body(*refs, cout):
    xp_ref, deg_ref, dinv_ref = refs[0], refs[1], refs[2]
    outs = refs[3:]
    dinv = lax.rsqrt(deg_ref[...] + 1.0)
    dinv_ref[...] = dinv
    xn = xp_ref[...] * dinv
    for c in range(cout):
        outs[c][...] = xn[:, c * _FC:(c + 1) * _FC]


def _xn0(x_pad, deg):
    np_rows, fpad = x_pad.shape
    cout = fpad // _FC
    bm = 1024
    grid = (np_rows // bm,)
    chunk_spec = pl.BlockSpec((bm, _FC), lambda i: (i, 0))
    res = pl.pallas_call(
        functools.partial(_xn0_body, cout=cout),
        grid=grid,
        in_specs=[
            pl.BlockSpec((bm, fpad), lambda i: (i, 0)),
            pl.BlockSpec((bm, 1), lambda i: (i, 0)),
        ],
        out_specs=[pl.BlockSpec((bm, 1), lambda i: (i, 0))] + [chunk_spec] * cout,
        out_shape=[jax.ShapeDtypeStruct((np_rows, 1), jnp.float32)]
        + [jax.ShapeDtypeStruct((np_rows, _FC), jnp.float32)] * cout,
    )(x_pad, deg)
    return res[0], list(res[1:])


def _layer_body(*refs, cin, cout, full_out):
    dinv_ref, w_ref, b_ref = refs[:3]
    sums = refs[3:3 + cin]
    xns = refs[3 + cin:3 + 2 * cin]
    outs = refs[3 + 2 * cin:]
    dinv = dinv_ref[...]
    xb = jnp.concatenate([s[...] + x[...] for s, x in zip(sums, xns)], axis=1)
    xb = xb * dinv
    h = jnp.dot(xb, w_ref[...], preferred_element_type=jnp.float32,
                precision=lax.Precision.HIGHEST)
    h = jnp.maximum(h + b_ref[...], 0.0)
    if full_out:
        outs[0][...] = h
    else:
        for c in range(cout):
            outs[c][...] = h[:, c * _FC:(c + 1) * _FC] * dinv


def _layer(dinv, wp, bp, sums, xns, cout=0, full_out=False):
    np_rows = dinv.shape[0]
    cin = len(sums)
    kin = wp.shape[0]
    nout = wp.shape[1]
    bm = 1024
    grid = (np_rows // bm,)
    chunk_spec = pl.BlockSpec((bm, _FC), lambda i: (i, 0))
    if full_out:
        out_specs = [pl.BlockSpec((bm, nout), lambda i: (i, 0))]
        out_shape = [jax.ShapeDtypeStruct((np_rows, nout), jnp.float32)]
    else:
        out_specs = [chunk_spec] * cout
        out_shape = [jax.ShapeDtypeStruct((np_rows, _FC), jnp.float32)] * cout
    res = pl.pallas_call(
        functools.partial(_layer_body, cin=cin, cout=cout, full_out=full_out),
        grid=grid,
        in_specs=[
            pl.BlockSpec((bm, 1), lambda i: (i, 0)),
            pl.BlockSpec((kin, nout), lambda i: (0, 0)),
            pl.BlockSpec((1, nout), lambda i: (0, 0)),
        ] + [chunk_spec] * (2 * cin),
        out_specs=out_specs,
        out_shape=out_shape,
    )(dinv, wp, bp, *sums, *xns)
    return list(res)


def _segmax_body(idsv_ref, h_ref, o_ref, *, n_valid):
    pid = pl.program_id(0)

    @pl.when(pid == 0)
    def _():
        o_ref[...] = jnp.zeros_like(o_ref)

    ids = idsv_ref[...]
    rowidx = pid * 128 + lax.broadcasted_iota(jnp.int32, (128, 1), 0)
    h = jnp.where(rowidx < n_valid, h_ref[...], 0.0)
    base_id = (idsv_ref[0, 0] // 8) * 8
    last_id = idsv_ref[127, 0]
    nwin = (last_id - base_id) // 8 + 1

    def wbody(i, carry):
        wb = pl.multiple_of(base_id + i * 8, 8)
        cols = []
        for w in range(8):
            m = ids == (wb + w)
            cols.append(jnp.max(jnp.where(m, h, 0.0), axis=0, keepdims=True))
        win = jnp.concatenate(cols, axis=0)
        o_ref[pl.ds(wb, 8), :] = jnp.maximum(o_ref[pl.ds(wb, 8), :], win)
        return carry

    lax.fori_loop(0, nwin, wbody, 0)


def _segmax(h3, idsv, n_valid, n_batch):
    np_rows, nf = h3.shape
    grid = (np_rows // 128,)
    out_rows = n_batch + 8
    return pl.pallas_call(
        functools.partial(_segmax_body, n_valid=n_valid),
        grid=grid,
        in_specs=[
            pl.BlockSpec((128, 1), lambda i: (i, 0)),
            pl.BlockSpec((128, nf), lambda i: (i, 0)),
        ],
        out_specs=pl.BlockSpec((out_rows, nf), lambda i: (0, 0)),
        out_shape=jax.ShapeDtypeStruct((out_rows, nf), jnp.float32),
    )(idsv, h3)


def _mm_body(x_ref, w_ref, b_ref, o_ref, *, act, pre):
    x = x_ref[...]
    if pre == "l2norm":
        nrm = jnp.sqrt(jnp.sum(x * x, axis=1, keepdims=True))
        x = x / jnp.maximum(nrm, 1e-12)
    acc = jnp.dot(x, w_ref[...], preferred_element_type=jnp.float32,
                  precision=lax.Precision.HIGHEST)
    acc = acc + b_ref[...]
    if act == "relu":
        acc = jnp.maximum(acc, 0.0)
    o_ref[...] = acc


def _mm(x, w, b, act="none", pre="none", block_m=1024):
    m, k = x.shape
    n = w.shape[1]
    bm = min(block_m, m)
    grid = (pl.cdiv(m, bm),)
    return pl.pallas_call(
        functools.partial(_mm_body, act=act, pre=pre),
        grid=grid,
        in_specs=[
            pl.BlockSpec((bm, k), lambda i: (i, 0)),
            pl.BlockSpec((k, n), lambda i: (0, 0)),
            pl.BlockSpec((1, n), lambda i: (0, 0)),
        ],
        out_specs=pl.BlockSpec((bm, n), lambda i: (i, 0)),
        out_shape=jax.ShapeDtypeStruct((m, n), jnp.float32),
    )(x, w, b.reshape(1, n))


def _prep_edges(ei, np_rows):
    e = ei.shape[1]
    unit = 128 * _NS * _G
    ep = pl.cdiv(e, unit) * unit
    src = jnp.concatenate(
        [ei[0], jnp.zeros((ep - e,), jnp.int32)]).reshape(-1, 128)
    dst = jnp.concatenate(
        [ei[1], jnp.full((ep - e,), np_rows - 1, jnp.int32)]).reshape(-1, 128)
    return src, dst


def _padw(w, rows, cols):
    return jnp.pad(w, ((0, rows - w.shape[0]), (0, cols - w.shape[1])))


def kernel(x1, edge_index1, batch1, cell, x2, edge_index2, batch2, W1, b1, W2, b2, W3, b3, Wg1, bg1, Wg2, bg2, Wr1, br1, Wr2, br2, Wr3, br3, Wf1, bf1, Wf2, bf2, Wf3, bf3, Wo, bo):
    n = x1.shape[0]
    nB = cell.shape[0]
    np_rows = _pad_rows(n)
    f_in = x1.shape[1]
    c1 = _nchunks(f_in)
    f1p = c1 * _FC
    c2 = _nchunks(W2.shape[1])
    f2p = c2 * _FC
    f3p = _nchunks(W3.shape[1]) * _FC

    src1, dst1 = _prep_edges(edge_index1, np_rows)
    src2, dst2 = _prep_edges(edge_index2, np_rows)
    x1p = jnp.pad(x1, ((0, np_rows - n), (0, f1p - f_in)))
    x2p = jnp.pad(x2, ((0, np_rows - n), (0, f1p - f_in)))
    idsv1 = jnp.pad(batch1, (0, np_rows - n), constant_values=nB - 1).reshape(np_rows, 1)
    idsv2 = jnp.pad(batch2, (0, np_rows - n), constant_values=nB - 1).reshape(np_rows, 1)

    slab = np_rows // _NS
    ones_col = jnp.ones((128, 1), jnp.float32)
    zeros_col = jnp.zeros((slab, 1), jnp.float32)
    zeros_slab = jnp.zeros((slab, _FC), jnp.float32)

    W1p = _padw(W1, f1p, f1p)
    b1p = _padw(b1.reshape(1, -1), 1, f1p)
    W2p = _padw(W2, f1p, f2p)
    b2p = _padw(b2.reshape(1, -1), 1, f2p)
    W3p = _padw(W3, f2p, f3p)
    b3p = _padw(b3.reshape(1, -1), 1, f3p)
    Wg1p = _padw(Wg1, f3p, Wg1.shape[1])

    deg1, deg2 = _sc_degrees(dst1, dst2, ones_col, zeros_col, np_rows)
    dinv1, xn1a = _xn0(x1p, deg1)
    dinv2, xn1b = _xn0(x2p, deg2)

    def agg_pair(xa_chunks, xb_chunks):
        jobs = [(c, src1, dst1) for c in xa_chunks] + \
               [(c, src2, dst2) for c in xb_chunks]
        outs = _sc_agg(jobs, zeros_slab, np_rows)
        k = len(xa_chunks)
        return outs[:k], outs[k:]

    s1a, s1b = agg_pair(xn1a, xn1b)
    xn2a = _layer(dinv1, W1p, b1p, s1a, xn1a, cout=c1)
    xn2b = _layer(dinv2, W1p, b1p, s1b, xn1b, cout=c1)
    s2a, s2b = agg_pair(xn2a, xn2b)
    xn3a = _layer(dinv1, W2p, b2p, s2a, xn2a, cout=c2)
    xn3b = _layer(dinv2, W2p, b2p, s2b, xn2b, cout=c2)
    s3a, s3b = agg_pair(xn3a, xn3b)
    (h3a,) = _layer(dinv1, W3p, b3p, s3a, xn3a, full_out=True)
    (h3b,) = _layer(dinv2, W3p, b3p, s3b, xn3b, full_out=True)

    ga = _segmax(h3a, idsv1, n, nB)[:nB]
    gb = _segmax(h3b, idsv2, n, nB)[:nB]
    ga = _mm(ga, Wg1p, bg1, act="relu", block_m=512)
    ga = _mm(ga, Wg2, bg2, block_m=512)
    gb = _mm(gb, Wg1p, bg1, act="relu", block_m=512)
    gb = _mm(gb, Wg2, bg2, block_m=512)

    cv = _mm(cell, Wr1, br1, act="relu", pre="l2norm", block_m=512)
    cv = _mm(cv, Wr2, br2, act="relu", block_m=512)
    cv = _mm(cv, Wr3, br3, act="relu", block_m=512)
    xc = jnp.concatenate([ga, gb, cv], axis=1)
    xc = _mm(xc, Wf1, bf1, act="relu", block_m=512)
    xc = _mm(xc, Wf2, bf2, act="relu", block_m=512)
    xc = _mm(xc, Wf3, bf3, act="relu", block_m=512)
    return _mm(xc, Wo, bo, block_m=512)
